# baseline (speedup 1.0000x reference)
import functools

import jax
import jax.numpy as jnp
from jax import lax
from jax.experimental import pallas as pl
from jax.experimental.pallas import tpu as pltpu
from jax.experimental.pallas import tpu_sc as plsc

_L = 16


def _row_argmax(score_ref, r, T):
    steps = T // _L
    lane_iota = lax.iota(jnp.int32, _L)
    neg_inf = jnp.full((_L,), -jnp.inf, dtype=jnp.float32)

    def body(j, carry):
        maxv, besti, curi = carry
        x = score_ref[r, pl.ds(j * _L, _L)]
        gt = x > maxv
        maxv = jnp.where(gt, x, maxv)
        besti = jnp.where(gt, curi, besti)
        return maxv, besti, curi + _L

    maxv, besti, _ = lax.fori_loop(
        0, steps, body, (neg_inf, lane_iota, lane_iota), unroll=8
    )
    m = jnp.max(maxv)
    masked = jnp.where(maxv == m, besti, jnp.int32(2147483647))
    return jnp.min(masked)


def _lane_extract(vec, lane):
    sel = lax.iota(jnp.int32, _L) == lane
    return jnp.sum(jnp.where(sel, vec, jnp.zeros_like(vec)))


def _sc_kernel_body(NC, B, T, ML, RPW,
                    score_hbm, off_hbm, frames_hbm, out_hbm,
                    sbuf, obuf, fbuf, outbuf,
                    sem_a, sem_b, sem_o, sem_f):
    wid = lax.axis_index("s") * NC + lax.axis_index("c")
    base = wid * RPW
    sems = (sem_a, sem_b)

    def _al(x, n=16):
        return pl.multiple_of(x, n)

    copies = [
        pltpu.async_copy(score_hbm.at[pl.ds(_al((base + r) * T), T)],
                         sbuf.at[r], sems[r])
        for r in range(2)
    ]
    inds = []
    for r in range(RPW):
        copies[r % 2].wait()
        inds.append(_row_argmax(sbuf, r % 2, T))
        if r + 2 < RPW:
            copies[r % 2] = pltpu.async_copy(
                score_hbm.at[pl.ds(_al((base + r + 2) * T), T)],
                sbuf.at[r % 2], sems[r % 2]
            )

    obase = [(inds[r] * 2) & jnp.int32(-16) for r in range(RPW)]
    ocopies = [
        pltpu.async_copy(
            off_hbm.at[pl.ds(_al((base + r) * 2 * T + obase[r]), _L)],
            obuf.at[r], sem_o
        )
        for r in range(RPW)
    ]
    for c in ocopies:
        c.wait()

    s_idx = []
    e_idx = []
    for r in range(RPW):
        ind = inds[r]
        lane = ind * 2 - obase[r]
        off0 = _lane_extract(obuf[r], lane)
        off1 = _lane_extract(obuf[r], lane + 1)
        center = (ind >> 2).astype(jnp.float32)
        half = (jnp.int32(4) << (ind & 3)).astype(jnp.float32)
        sf = (center - half) + off0
        ef = (center + half) + off1
        s = sf.astype(jnp.int32)
        s = s - (s.astype(jnp.float32) > sf).astype(jnp.int32)
        e = ef.astype(jnp.int32)
        e = e - (e.astype(jnp.float32) > ef).astype(jnp.int32)
        s_idx.append(jnp.clip(s, 0, ML - 1))
        e_idx.append(jnp.clip(e, 0, ML - 1))

    fbase = [(v & jnp.int32(-16)) for v in s_idx] + [
        (v & jnp.int32(-16)) for v in e_idx
    ]
    fcopies = [
        pltpu.async_copy(
            frames_hbm.at[pl.ds(_al((base + (k % RPW)) * ML + fbase[k]), _L)],
            fbuf.at[k],
            sem_f,
        )
        for k in range(2 * RPW)
    ]
    for c in fcopies:
        c.wait()

    lane_iota = lax.iota(jnp.int32, _L)
    acc = jnp.zeros((_L,), dtype=jnp.float32)
    for r in range(RPW):
        f0 = _lane_extract(fbuf[r], s_idx[r] - fbase[r])
        f1 = _lane_extract(fbuf[RPW + r], e_idx[r] - fbase[RPW + r])
        acc = jnp.where(lane_iota == 2 * r, f0.astype(jnp.float32), acc)
        acc = jnp.where(lane_iota == 2 * r + 1,
                        f1.astype(jnp.float32) + 1.0, acc)
    outbuf[...] = acc
    pltpu.sync_copy(outbuf.at[pl.ds(0, 2 * RPW)],
                    out_hbm.at[pl.ds(_al(base * 2, 8), 2 * RPW)])


def kernel(temporal_score, temporal_offset, frames_id):
    B, T = temporal_score.shape
    ML = frames_id.shape[1]
    info = plsc.get_sparse_core_info()
    NC, NS = info.num_cores, info.num_subcores
    NW = NC * NS
    RPW = B // NW

    score_flat = temporal_score.reshape(B * T)
    off_flat = temporal_offset.reshape(B * T * 2)
    frames_flat = frames_id.reshape(B * ML)

    mesh = plsc.VectorSubcoreMesh(core_axis_name="c", subcore_axis_name="s")
    k = functools.partial(
        pl.kernel,
        mesh=mesh,
        out_type=jax.ShapeDtypeStruct((B * 2,), jnp.float32),
        scratch_types=[
            pltpu.VMEM((2, T), jnp.float32),
            pltpu.VMEM((RPW, _L), jnp.float32),
            pltpu.VMEM((2 * RPW, _L), jnp.int32),
            pltpu.VMEM((_L,), jnp.float32),
            pltpu.SemaphoreType.DMA,
            pltpu.SemaphoreType.DMA,
            pltpu.SemaphoreType.DMA,
            pltpu.SemaphoreType.DMA,
        ],
        compiler_params=pltpu.CompilerParams(
            use_tc_tiling_on_sc=False, needs_layout_passes=False
        ),
    )(functools.partial(_sc_kernel_body, NC, B, T, ML, RPW))
    return k(score_flat, off_flat, frames_flat).reshape(B, 2)


# speedup vs baseline: 47.1540x; 47.1540x over previous
import functools

import jax
import jax.numpy as jnp
from jax import lax
from jax.experimental import pallas as pl
from jax.experimental.pallas import tpu as pltpu
from jax.experimental.pallas import tpu_sc as plsc

_L = 16


def _al(x, n=16):
    return pl.multiple_of(x, n)


def _lane_extract_f(vec, lane):
    sel = lax.iota(jnp.int32, _L) == lane
    return jnp.sum(jnp.where(sel, vec, jnp.zeros_like(vec)))


def _half_scan(sbuf, t0, t1, col0, carry):
    def body(t, carry):
        maxs, bests, curi = carry
        maxs = list(maxs)
        bests = list(bests)
        for k in range(128 // _L):
            for r in range(8):
                x = sbuf[t, r, pl.ds(k * _L, _L)]
                gt = x > maxs[r]
                maxs[r] = jnp.where(gt, x, maxs[r])
                bests[r] = jnp.where(gt, curi, bests[r])
            curi = curi + _L
        return tuple(maxs), tuple(bests), curi

    return lax.fori_loop(t0, t1, body, carry, unroll=2)


def _sc_kernel_body(NS, B, T, ML, RPW,
                    score_hbm, off_hbm, frames_hbm, out_hbm,
                    sbuf, obuf, fbuf, outbuf, vbuf, shmax, shidx,
                    sem_a, sem_b, osems, sem_f, sem_m):
    cid = lax.axis_index("c")
    sid = lax.axis_index("s")
    g = cid * (NS // 2) + (sid >> 1)
    h = sid & 1
    base = g * 8 + h * RPW
    HT = T // 512

    ca = pltpu.async_copy(score_hbm.at[g, pl.ds(_al(h * 2 * HT), HT)],
                          sbuf.at[pl.ds(0, HT)], sem_a)
    cb = pltpu.async_copy(score_hbm.at[g, pl.ds(_al(h * 2 * HT + HT), HT)],
                          sbuf.at[pl.ds(HT, HT)], sem_b)
    lane_iota = lax.iota(jnp.int32, _L)
    neg_inf = jnp.full((_L,), -jnp.inf, dtype=jnp.float32)
    col0 = lane_iota + h * (T // 2)
    carry = ((neg_inf,) * 8, (col0,) * 8, col0)
    with jax.named_scope("waitA"):
        ca.wait()
    with jax.named_scope("scanA"):
        carry = _half_scan(sbuf, 0, HT, col0, carry)
    with jax.named_scope("waitB"):
        cb.wait()
    with jax.named_scope("scanB"):
        maxs, bests, _ = _half_scan(sbuf, HT, 2 * HT, col0, carry)

    with jax.named_scope("reduce_pack"):
        maxvec = jnp.zeros((_L,), dtype=jnp.float32)
        idxvec = jnp.zeros((_L,), dtype=jnp.int32)
        for r in range(8):
            m = jnp.max(maxs[r])
            i = jnp.min(jnp.where(maxs[r] == m, bests[r],
                                  jnp.int32(2147483647)))
            maxvec = jnp.where(lane_iota == r, m, maxvec)
            idxvec = jnp.where(lane_iota == r, i, idxvec)
        vbuf[...] = maxvec
        pltpu.sync_copy(vbuf, shmax.at[sid])
        fbuf[0, ...] = idxvec
        pltpu.sync_copy(fbuf.at[0], shidx.at[sid])
    plsc.subcore_barrier()

    with jax.named_scope("merge_argmax"):
        pltpu.async_copy(shmax.at[sid + 1 - 2 * h], vbuf, sem_m).wait()
        pltpu.async_copy(shidx.at[sid + 1 - 2 * h], fbuf.at[0], sem_m).wait()
        pmax = vbuf[...]
        pidx = fbuf[0, ...]
        better = (pmax > maxvec) | ((pmax == maxvec) & (pidx < idxvec))
        gidx = jnp.where(better, pidx, idxvec)
        inds = [jnp.sum(jnp.where(lane_iota == h * RPW + r, gidx,
                                  jnp.zeros_like(gidx)))
                for r in range(RPW)]

    with jax.named_scope("gather_phase"):
        ocopies = []
        for r in range(RPW):
            ind = inds[r]
            for comp in range(2):
                ocopies.append(pltpu.async_copy(
                    off_hbm.at[base + r, ind >> 7, comp,
                               pl.ds(_al((ind & 127) & ~15), _L)],
                    obuf.at[2 * r + comp], osems[r],
                ))

        s_idx, e_idx, fcopies = [], [], []
        for r in range(RPW):
            ocopies[2 * r].wait()
            ocopies[2 * r + 1].wait()
            ind = inds[r]
            lane = ind & 15
            off0 = _lane_extract_f(obuf[2 * r], lane)
            off1 = _lane_extract_f(obuf[2 * r + 1], lane)
            center = (ind >> 2).astype(jnp.float32)
            half = (jnp.int32(4) << (ind & 3)).astype(jnp.float32)
            sf = (center - half) + off0
            ef = (center + half) + off1
            s = sf.astype(jnp.int32)
            s = s - (s.astype(jnp.float32) > sf).astype(jnp.int32)
            e = ef.astype(jnp.int32)
            e = e - (e.astype(jnp.float32) > ef).astype(jnp.int32)
            s_idx.append(jnp.clip(s, 0, ML - 1))
            e_idx.append(jnp.clip(e, 0, ML - 1))
            for k, v in ((2 * r, s_idx[r]), (2 * r + 1, e_idx[r])):
                fcopies.append(pltpu.async_copy(
                    frames_hbm.at[g, v >> 7, h * RPW + r,
                                  pl.ds(_al((v & 127) & ~15), _L)],
                    fbuf.at[1 + k], sem_f,
                ))
        for c in fcopies:
            c.wait()

    with jax.named_scope("pack_merge"):
        acc = jnp.zeros((_L,), dtype=jnp.float32)
        for r in range(RPW):
            f0 = _lane_extract_f(fbuf[1 + 2 * r].astype(jnp.float32),
                                 s_idx[r] & 15)
            f1 = _lane_extract_f(fbuf[2 + 2 * r].astype(jnp.float32),
                                 e_idx[r] & 15)
            acc = jnp.where(lane_iota == h * RPW + r, f0, acc)
            acc = jnp.where(lane_iota == 8 + h * RPW + r, f1 + 1.0, acc)

        @pl.when(h == 1)
        def _():
            outbuf[...] = acc
            pltpu.sync_copy(outbuf, shmax.at[sid])

        plsc.subcore_barrier()

        @pl.when(h == 0)
        def _():
            pltpu.async_copy(shmax.at[sid + 1], outbuf, sem_m).wait()
            outbuf[...] = acc + outbuf[...]
            b0 = g * 8
            c1 = pltpu.async_copy(outbuf.at[pl.ds(0, 8)],
                                  out_hbm.at[pl.ds(_al(b0, 8), 8)], sem_m)
            c2 = pltpu.async_copy(outbuf.at[pl.ds(8, 8)],
                                  out_hbm.at[pl.ds(_al(B + b0, 8), 8)],
                                  sem_m)
            c1.wait()
            c2.wait()


def kernel(temporal_score, temporal_offset, frames_id):
    B, T = temporal_score.shape
    ML = frames_id.shape[1]
    info = plsc.get_sparse_core_info()
    NC, NS = info.num_cores, info.num_subcores
    NW = NC * NS
    RPW = B // NW

    score_po = temporal_score.reshape(B // 8, 8, T // 128, 128).transpose(
        0, 2, 1, 3)
    off_po = temporal_offset.reshape(B, T // 128, 128, 2).transpose(
        0, 1, 3, 2)
    frames_po = frames_id.reshape(B // 8, 8, ML // 128, 128).transpose(
        0, 2, 1, 3)

    mesh = plsc.VectorSubcoreMesh(core_axis_name="c", subcore_axis_name="s")
    k = functools.partial(
        pl.kernel,
        mesh=mesh,
        out_type=jax.ShapeDtypeStruct((2 * B,), jnp.float32),
        scratch_types=[
            pltpu.VMEM((T // 256, 8, 128), jnp.float32),
            pltpu.VMEM((2 * RPW, _L), jnp.float32),
            pltpu.VMEM((1 + 2 * RPW, _L), jnp.int32),
            pltpu.VMEM((_L,), jnp.float32),
            pltpu.VMEM((_L,), jnp.float32),
            pltpu.VMEM_SHARED((NS, _L), jnp.float32),
            pltpu.VMEM_SHARED((NS, _L), jnp.int32),
            pltpu.SemaphoreType.DMA,
            pltpu.SemaphoreType.DMA,
            [pltpu.SemaphoreType.DMA] * RPW,
            pltpu.SemaphoreType.DMA,
            pltpu.SemaphoreType.DMA,
        ],
        compiler_params=pltpu.CompilerParams(
            use_tc_tiling_on_sc=False, needs_layout_passes=False
        ),
    )(functools.partial(_sc_kernel_body, NS, B, T, ML, RPW))
    return k(score_po, off_po, frames_po).reshape(2, B).T


# speedup vs baseline: 55.4357x; 1.1756x over previous
import functools

import jax
import jax.numpy as jnp
from jax import lax
from jax.experimental import pallas as pl
from jax.experimental.pallas import tpu as pltpu
from jax.experimental.pallas import tpu_sc as plsc

_L = 16


def _al(x, n=16):
    return pl.multiple_of(x, n)


def _row_argmax(score_ref, buf, T):
    lane_iota = lax.iota(jnp.int32, _L)
    neg_inf = jnp.full((_L,), -jnp.inf, dtype=jnp.float32)
    half_t = T // 256

    def body(t, carry):
        maxa, besta, curia, maxb, bestb, curib = carry
        for k in range(128 // _L):
            xa = score_ref[buf, t, pl.ds(k * _L, _L)]
            xb = score_ref[buf, half_t + t, pl.ds(k * _L, _L)]
            ga = xa > maxa
            gb = xb > maxb
            maxa = jnp.where(ga, xa, maxa)
            besta = jnp.where(ga, curia, besta)
            maxb = jnp.where(gb, xb, maxb)
            bestb = jnp.where(gb, curib, bestb)
            curia = curia + _L
            curib = curib + _L
        return maxa, besta, curia, maxb, bestb, curib

    ib0 = lane_iota + T // 2
    maxa, besta, _, maxb, bestb, _ = lax.fori_loop(
        0, half_t, body, (neg_inf, lane_iota, lane_iota, neg_inf, ib0, ib0),
        unroll=2,
    )
    gb = maxb > maxa
    maxv = jnp.where(gb, maxb, maxa)
    besti = jnp.where(gb, bestb, besta)
    m = jnp.max(maxv)
    masked = jnp.where(maxv == m, besti, jnp.int32(2147483647))
    return jnp.min(masked)


def _lane_extract(vec, lane):
    sel = lax.iota(jnp.int32, _L) == lane
    return jnp.sum(jnp.where(sel, vec, jnp.zeros_like(vec)))


def _sc_kernel_body(NS, B, T, ML, RPW,
                    score_hbm, off_hbm, frames_hbm, out_hbm,
                    sbuf, obuf, fbuf, outbuf, shared,
                    sems, osems, sem_f, sem_m):
    cid = lax.axis_index("c")
    sid = lax.axis_index("s")
    wid = cid * NS + sid
    base = wid * RPW
    g = base // 8

    copies = [
        pltpu.async_copy(score_hbm.at[g, :, base % 8 + r], sbuf.at[r],
                         sems[r])
        for r in range(RPW)
    ]
    inds, ocopies = [], []
    for r in range(RPW):
        copies[r].wait()
        ind = _row_argmax(sbuf, r, T)
        inds.append(ind)
        for comp in range(2):
            ocopies.append(pltpu.async_copy(
                off_hbm.at[base + r, ind >> 7, comp,
                           pl.ds(_al((ind & 127) & ~15), _L)],
                obuf.at[2 * r + comp], osems[r],
            ))

    s_idx, e_idx, fcopies = [], [], []
    for r in range(RPW):
        ocopies[2 * r].wait()
        ocopies[2 * r + 1].wait()
        ind = inds[r]
        lane = ind & 15
        off0 = _lane_extract(obuf[2 * r], lane)
        off1 = _lane_extract(obuf[2 * r + 1], lane)
        center = (ind >> 2).astype(jnp.float32)
        half = (jnp.int32(4) << (ind & 3)).astype(jnp.float32)
        sf = (center - half) + off0
        ef = (center + half) + off1
        s = sf.astype(jnp.int32)
        s = s - (s.astype(jnp.float32) > sf).astype(jnp.int32)
        e = ef.astype(jnp.int32)
        e = e - (e.astype(jnp.float32) > ef).astype(jnp.int32)
        s_idx.append(jnp.clip(s, 0, ML - 1))
        e_idx.append(jnp.clip(e, 0, ML - 1))
        for k, v in ((r, s_idx[r]), (RPW + r, e_idx[r])):
            fcopies.append(pltpu.async_copy(
                frames_hbm.at[g, v >> 7, base % 8 + r,
                              pl.ds(_al((v & 127) & ~15), _L)],
                fbuf.at[k], sem_f,
            ))
    for c in fcopies:
        c.wait()

    lane_iota = lax.iota(jnp.int32, _L)
    acc = jnp.zeros((_L,), dtype=jnp.float32)
    for r in range(RPW):
        f0 = _lane_extract(fbuf[r].astype(jnp.float32), s_idx[r] & 15)
        f1 = _lane_extract(fbuf[RPW + r].astype(jnp.float32), e_idx[r] & 15)
        acc = jnp.where(lane_iota == base % 8 + r, f0, acc)
        acc = jnp.where(lane_iota == 8 + base % 8 + r, f1 + 1.0, acc)

    @pl.when(sid % 2 == 1)
    def _():
        outbuf[...] = acc
        pltpu.sync_copy(outbuf, shared.at[sid])

    plsc.subcore_barrier()

    @pl.when(sid % 2 == 0)
    def _():
        pltpu.async_copy(shared.at[sid + 1], outbuf, sem_m).wait()
        outbuf[...] = acc + outbuf[...]
        b0 = base
        c1 = pltpu.async_copy(outbuf.at[pl.ds(0, 8)],
                              out_hbm.at[pl.ds(_al(b0, 8), 8)], sem_m)
        c2 = pltpu.async_copy(outbuf.at[pl.ds(8, 8)],
                              out_hbm.at[pl.ds(_al(B + b0, 8), 8)], sem_m)
        c1.wait()
        c2.wait()


def kernel(temporal_score, temporal_offset, frames_id):
    B, T = temporal_score.shape
    ML = frames_id.shape[1]
    info = plsc.get_sparse_core_info()
    NC, NS = info.num_cores, info.num_subcores
    NW = NC * NS
    RPW = B // NW

    score_po = temporal_score.reshape(B // 8, 8, T // 128, 128).transpose(
        0, 2, 1, 3)
    off_po = temporal_offset.reshape(B, T // 128, 128, 2).transpose(
        0, 1, 3, 2)
    frames_po = frames_id.reshape(B // 8, 8, ML // 128, 128).transpose(
        0, 2, 1, 3)

    mesh = plsc.VectorSubcoreMesh(core_axis_name="c", subcore_axis_name="s")
    k = functools.partial(
        pl.kernel,
        mesh=mesh,
        out_type=jax.ShapeDtypeStruct((2 * B,), jnp.float32),
        scratch_types=[
            pltpu.VMEM((RPW, T // 128, 128), jnp.float32),
            pltpu.VMEM((2 * RPW, _L), jnp.float32),
            pltpu.VMEM((2 * RPW, _L), jnp.int32),
            pltpu.VMEM((_L,), jnp.float32),
            pltpu.VMEM_SHARED((NS, _L), jnp.float32),
            [pltpu.SemaphoreType.DMA] * RPW,
            [pltpu.SemaphoreType.DMA] * RPW,
            pltpu.SemaphoreType.DMA,
            pltpu.SemaphoreType.DMA,
        ],
        compiler_params=pltpu.CompilerParams(
            use_tc_tiling_on_sc=False, needs_layout_passes=False
        ),
    )(functools.partial(_sc_kernel_body, NS, B, T, ML, RPW))
    return k(score_po, off_po, frames_po).reshape(2, B).T
